# baseline (speedup 1.0000x reference)
import jax
import jax.numpy as jnp
from jax import lax
from jax.experimental import pallas as pl
from jax.experimental.pallas import tpu as pltpu
from jax.experimental.pallas import tpu_sc as plsc

VOCAB_N = 1000000
DIM_N = 64
HID_N = 512
B_N = 16384

_NC = 2
_NS = 16
_NW = _NC * _NS
_BPW = B_N // _NW
_NG = _BPW // 16

_SEGS = 8192
_S0 = 3840
_SPT = _S0 // _NW
_CHUNK = 384
_NCHUNK = _SPT * 128 // _CHUNK
_NBUF = 4

_VBLK = 32768
_NVB = (VOCAB_N - _S0 * 128 + _VBLK - 1) // _VBLK
_B0OFF = _S0 * 128 // _VBLK


def _b0_body(tt_ref, wi_ref, out_ref):
    row = jnp.dot(wi_ref[...].T, tt_ref[...], preferred_element_type=jnp.float32)
    out_ref[...] = row.reshape(_VBLK // 128, 128)


def _tc_table_dot(table_t, w_i):
    return pl.pallas_call(
        _b0_body,
        grid=(_NVB,),
        in_specs=[
            pl.BlockSpec((DIM_N, _VBLK), lambda i: (0, i + _B0OFF)),
            pl.BlockSpec((DIM_N, 1), lambda i: (0, 0)),
        ],
        out_specs=pl.BlockSpec((_VBLK // 128, 128), lambda i: (i + _B0OFF, 0)),
        out_shape=jax.ShapeDtypeStruct((_SEGS, 128), jnp.float32),
    )(table_t, w_i)


def _sc1_body(table_t, wbc, vout, bufs_v, wbc_v, stage_v, sems):
    wid = lax.axis_index("s") * _NC + lax.axis_index("c")
    c0 = wid * (_SPT * 128)
    pltpu.sync_copy(wbc, wbc_v)

    for b in range(_NBUF):
        pltpu.async_copy(
            table_t.at[:, pl.ds(c0 + b * _CHUNK, _CHUNK)], bufs_v.at[b], sems.at[b]
        )

    nj = _CHUNK // 16

    def one_chunk(k, b):
        pltpu.make_async_copy(
            table_t.at[:, pl.ds(0, _CHUNK)], bufs_v.at[b], sems.at[b]
        ).wait()

        def cbody(ci, accs):
            c = ci * 2
            wv0 = wbc_v[c, :]
            wv1 = wbc_v[c + 1, :]
            accs = tuple(
                accs[j] + wv0 * bufs_v[b, c, pl.ds(j * 16, 16)] for j in range(nj)
            )
            return tuple(
                accs[j] + wv1 * bufs_v[b, c + 1, pl.ds(j * 16, 16)]
                for j in range(nj)
            )

        accs = lax.fori_loop(
            0, DIM_N // 2, cbody,
            tuple(jnp.zeros((16,), jnp.float32) for _ in range(nj)),
        )
        pltpu.async_copy(
            table_t.at[:, pl.ds(c0 + (k + _NBUF) * _CHUNK, _CHUNK)],
            bufs_v.at[b],
            sems.at[b],
        )
        for j in range(nj):
            row = k * (_CHUNK // 128) + j // 8
            stage_v[row, pl.ds((j % 8) * 16, 16)] = accs[j]

    def chunk_iter(k, carry):
        bdyn = lax.rem(k, _NBUF)
        for b in range(_NBUF):
            @pl.when(bdyn == b)
            def _(b=b):
                one_chunk(k, b)
        return carry

    lax.fori_loop(0, _NCHUNK, chunk_iter, 0)
    for b in range(_NBUF):
        pltpu.make_async_copy(
            table_t.at[:, pl.ds(0, _CHUNK)], bufs_v.at[b], sems.at[b]
        ).wait()
    pltpu.sync_copy(stage_v, vout.at[pl.ds(wid * _SPT, _SPT)])


def _sc_table_dot(table_t, wbc):
    mesh = plsc.VectorSubcoreMesh(core_axis_name="c", subcore_axis_name="s")
    return pl.kernel(
        _sc1_body,
        mesh=mesh,
        out_type=jax.ShapeDtypeStruct((_SEGS, 128), jnp.float32),
        scratch_types=[
            pltpu.VMEM((_NBUF, DIM_N, _CHUNK), jnp.float32),
            pltpu.VMEM((DIM_N, 16), jnp.float32),
            pltpu.VMEM((_SPT, 128), jnp.float32),
            pltpu.SemaphoreType.DMA((_NBUF,)),
        ],
        compiler_params=pltpu.CompilerParams(needs_layout_passes=False),
    )(table_t, wbc)


def _sc2_body(va, vb, xf, out, idx_v, seg_v, lane_v, da_v, db_v, s3_v, sem):
    wid = lax.axis_index("s") * _NC + lax.axis_index("c")
    rbase = wid * _BPW
    pltpu.sync_copy(xf.at[pl.ds(rbase, _BPW)], idx_v)

    def split(g, carry):
        r = idx_v[pl.ds(g * 16, 16)]
        seg_v[pl.ds(g * 16, 16)] = r >> 7
        lane_v[pl.ds(g * 16, 16)] = r & 127
        return carry

    lax.fori_loop(0, _NG, split, 0)

    lanes = lax.iota(jnp.int32, 16)
    for h in range(2):
        half = h * (_BPW // 2)
        cpa = pltpu.async_copy(va.at[seg_v.at[pl.ds(half, _BPW // 2)]], da_v, sem)
        cpb = pltpu.async_copy(vb.at[seg_v.at[pl.ds(half, _BPW // 2)]], db_v, sem)
        cpa.wait()
        cpb.wait()

        def extract(g, carry):
            rows = g * 16 + lanes
            cols = lane_v[pl.ds(half + g * 16, 16)]
            a = plsc.load_gather(da_v, [rows, cols])
            b = plsc.load_gather(db_v, [rows, cols])
            seg = seg_v[pl.ds(half + g * 16, 16)]
            s3_v[pl.ds(half + g * 16, 16)] = jnp.where(seg < _S0, a, b)
            return carry

        lax.fori_loop(0, _NG // 2, extract, 0)

    pltpu.sync_copy(s3_v, out.at[pl.ds(rbase, _BPW)])


def _sc_gather(va, vb, xf):
    mesh = plsc.VectorSubcoreMesh(core_axis_name="c", subcore_axis_name="s")
    return pl.kernel(
        _sc2_body,
        mesh=mesh,
        out_type=jax.ShapeDtypeStruct((B_N,), jnp.float32),
        scratch_types=[
            pltpu.VMEM((_BPW,), jnp.int32),
            pltpu.VMEM((_BPW,), jnp.int32),
            pltpu.VMEM((_BPW,), jnp.int32),
            pltpu.VMEM((_BPW // 2, 128), jnp.float32),
            pltpu.VMEM((_BPW // 2, 128), jnp.float32),
            pltpu.VMEM((_BPW,), jnp.float32),
            pltpu.SemaphoreType.DMA,
        ],
        compiler_params=pltpu.CompilerParams(needs_layout_passes=False),
    )(va, vb, xf)


_BLK = 2048


def _b1_body(dh_ref, cv_ref, ws_ref, wc_ref, out_ref):
    acc = jnp.dot(dh_ref[...], ws_ref[...], preferred_element_type=jnp.float32)
    acc = acc + jnp.dot(cv_ref[...], wc_ref[...], preferred_element_type=jnp.float32)
    out_ref[...] = acc.reshape(_BLK)


def _tc_dense(dec_hidden, context_vector, w_s, w_c):
    return pl.pallas_call(
        _b1_body,
        grid=(B_N // _BLK,),
        in_specs=[
            pl.BlockSpec((_BLK, HID_N), lambda i: (i, 0)),
            pl.BlockSpec((_BLK, HID_N), lambda i: (i, 0)),
            pl.BlockSpec((HID_N, 1), lambda i: (0, 0)),
            pl.BlockSpec((HID_N, 1), lambda i: (0, 0)),
        ],
        out_specs=pl.BlockSpec((_BLK,), lambda i: (i,)),
        out_shape=jax.ShapeDtypeStruct((B_N,), jnp.float32),
    )(dec_hidden, context_vector, w_s, w_c)


def _b2_body(s12_ref, s3_ref, bias_ref, out_ref):
    z = s12_ref[...] + s3_ref[...] + bias_ref[0, 0]
    out_ref[...] = 1.0 / (1.0 + jnp.exp(-z))


def _tc_combine(s12, s3, bias):
    return pl.pallas_call(
        _b2_body,
        grid=(B_N // _BLK,),
        in_specs=[
            pl.BlockSpec((_BLK,), lambda i: (i,)),
            pl.BlockSpec((_BLK,), lambda i: (i,)),
            pl.BlockSpec((1, 1), lambda i: (0, 0), memory_space=pltpu.SMEM),
        ],
        out_specs=pl.BlockSpec((_BLK,), lambda i: (i,)),
        out_shape=jax.ShapeDtypeStruct((B_N,), jnp.float32),
    )(s12, s3, bias)


def kernel(context_vector, dec_hidden, x, embedding_matrix, w_s, b_s, w_c, b_c, w_i, b_i):
    xf = x.reshape(B_N).astype(jnp.int32)
    table_t = embedding_matrix.T
    wbc = jnp.broadcast_to(w_i.reshape(DIM_N, 1), (DIM_N, 16))
    v_sc = _sc_table_dot(table_t, wbc)
    v_tc = _tc_table_dot(table_t, w_i)
    s3 = _sc_gather(v_sc, v_tc, xf)
    s12 = _tc_dense(dec_hidden, context_vector, w_s, w_c)
    bias = (b_s + b_c + b_i).reshape(1, 1)
    return _tc_combine(s12, s3, bias).reshape(B_N, 1)


# speedup vs baseline: 1.0149x; 1.0149x over previous
import jax
import jax.numpy as jnp
from jax import lax
from jax.experimental import pallas as pl
from jax.experimental.pallas import tpu as pltpu
from jax.experimental.pallas import tpu_sc as plsc

VOCAB_N = 1000000
DIM_N = 64
HID_N = 512
B_N = 16384

_NC = 2
_NS = 16
_NW = _NC * _NS
_BPW = B_N // _NW
_NG = _BPW // 16

_SEGS = 8192
_S0 = 3072
_SPT = _S0 // _NW
_CHUNK = 384
_NCHUNK = _SPT * 128 // _CHUNK
_NBUF = 4

_VBLK = 32768
_NVB = (VOCAB_N - _S0 * 128 + _VBLK - 1) // _VBLK
_B0OFF = _S0 * 128 // _VBLK


def _b0_body(tt_ref, wi_ref, out_ref):
    row = jnp.dot(wi_ref[...].T, tt_ref[...], preferred_element_type=jnp.float32)
    out_ref[...] = row.reshape(_VBLK // 128, 128)


def _tc_table_dot(table_t, w_i):
    return pl.pallas_call(
        _b0_body,
        grid=(_NVB,),
        in_specs=[
            pl.BlockSpec((DIM_N, _VBLK), lambda i: (0, i + _B0OFF)),
            pl.BlockSpec((DIM_N, 1), lambda i: (0, 0)),
        ],
        out_specs=pl.BlockSpec((_VBLK // 128, 128), lambda i: (i + _B0OFF, 0)),
        out_shape=jax.ShapeDtypeStruct((_SEGS, 128), jnp.float32),
    )(table_t, w_i)


def _sc1_body(table_t, wbc, vout, bufs_v, wbc_v, stage_v, sems):
    wid = lax.axis_index("s") * _NC + lax.axis_index("c")
    c0 = wid * (_SPT * 128)
    pltpu.sync_copy(wbc, wbc_v)

    for b in range(_NBUF):
        pltpu.async_copy(
            table_t.at[:, pl.ds(c0 + b * _CHUNK, _CHUNK)], bufs_v.at[b], sems.at[b]
        )

    nj = _CHUNK // 16

    def one_chunk(k, b):
        pltpu.make_async_copy(
            table_t.at[:, pl.ds(0, _CHUNK)], bufs_v.at[b], sems.at[b]
        ).wait()

        def cbody(ci, accs):
            c = ci * 2
            wv0 = wbc_v[c, :]
            wv1 = wbc_v[c + 1, :]
            accs = tuple(
                accs[j] + wv0 * bufs_v[b, c, pl.ds(j * 16, 16)] for j in range(nj)
            )
            return tuple(
                accs[j] + wv1 * bufs_v[b, c + 1, pl.ds(j * 16, 16)]
                for j in range(nj)
            )

        accs = lax.fori_loop(
            0, DIM_N // 2, cbody,
            tuple(jnp.zeros((16,), jnp.float32) for _ in range(nj)),
        )
        pltpu.async_copy(
            table_t.at[:, pl.ds(c0 + (k + _NBUF) * _CHUNK, _CHUNK)],
            bufs_v.at[b],
            sems.at[b],
        )
        for j in range(nj):
            row = k * (_CHUNK // 128) + j // 8
            stage_v[row, pl.ds((j % 8) * 16, 16)] = accs[j]

    def chunk_iter(k, carry):
        bdyn = lax.rem(k, _NBUF)
        for b in range(_NBUF):
            @pl.when(bdyn == b)
            def _(b=b):
                one_chunk(k, b)
        return carry

    lax.fori_loop(0, _NCHUNK, chunk_iter, 0)
    for b in range(_NBUF):
        pltpu.make_async_copy(
            table_t.at[:, pl.ds(0, _CHUNK)], bufs_v.at[b], sems.at[b]
        ).wait()
    pltpu.sync_copy(stage_v, vout.at[pl.ds(wid * _SPT, _SPT)])


def _sc_table_dot(table_t, wbc):
    mesh = plsc.VectorSubcoreMesh(core_axis_name="c", subcore_axis_name="s")
    return pl.kernel(
        _sc1_body,
        mesh=mesh,
        out_type=jax.ShapeDtypeStruct((_SEGS, 128), jnp.float32),
        scratch_types=[
            pltpu.VMEM((_NBUF, DIM_N, _CHUNK), jnp.float32),
            pltpu.VMEM((DIM_N, 16), jnp.float32),
            pltpu.VMEM((_SPT, 128), jnp.float32),
            pltpu.SemaphoreType.DMA((_NBUF,)),
        ],
        compiler_params=pltpu.CompilerParams(needs_layout_passes=False),
    )(table_t, wbc)


def _sc2_body(va, vb, xf, out, idx_v, seg_v, lane_v, da_v, db_v, s3_v, sem):
    wid = lax.axis_index("s") * _NC + lax.axis_index("c")
    rbase = wid * _BPW
    pltpu.sync_copy(xf.at[pl.ds(rbase, _BPW)], idx_v)

    def split(g, carry):
        r = idx_v[pl.ds(g * 16, 16)]
        seg_v[pl.ds(g * 16, 16)] = r >> 7
        lane_v[pl.ds(g * 16, 16)] = r & 127
        return carry

    lax.fori_loop(0, _NG, split, 0)

    lanes = lax.iota(jnp.int32, 16)
    for h in range(2):
        half = h * (_BPW // 2)
        cpa = pltpu.async_copy(va.at[seg_v.at[pl.ds(half, _BPW // 2)]], da_v, sem)
        cpb = pltpu.async_copy(vb.at[seg_v.at[pl.ds(half, _BPW // 2)]], db_v, sem)
        cpa.wait()
        cpb.wait()

        def extract(g, carry):
            rows = g * 16 + lanes
            cols = lane_v[pl.ds(half + g * 16, 16)]
            a = plsc.load_gather(da_v, [rows, cols])
            b = plsc.load_gather(db_v, [rows, cols])
            seg = seg_v[pl.ds(half + g * 16, 16)]
            s3_v[pl.ds(half + g * 16, 16)] = jnp.where(seg < _S0, a, b)
            return carry

        lax.fori_loop(0, _NG // 2, extract, 0)

    pltpu.sync_copy(s3_v, out.at[pl.ds(rbase, _BPW)])


def _sc_gather(va, vb, xf):
    mesh = plsc.VectorSubcoreMesh(core_axis_name="c", subcore_axis_name="s")
    return pl.kernel(
        _sc2_body,
        mesh=mesh,
        out_type=jax.ShapeDtypeStruct((B_N,), jnp.float32),
        scratch_types=[
            pltpu.VMEM((_BPW,), jnp.int32),
            pltpu.VMEM((_BPW,), jnp.int32),
            pltpu.VMEM((_BPW,), jnp.int32),
            pltpu.VMEM((_BPW // 2, 128), jnp.float32),
            pltpu.VMEM((_BPW // 2, 128), jnp.float32),
            pltpu.VMEM((_BPW,), jnp.float32),
            pltpu.SemaphoreType.DMA,
        ],
        compiler_params=pltpu.CompilerParams(needs_layout_passes=False),
    )(va, vb, xf)


_BLK = 2048


def _b1_body(dh_ref, cv_ref, ws_ref, wc_ref, out_ref):
    acc = jnp.dot(dh_ref[...], ws_ref[...], preferred_element_type=jnp.float32)
    acc = acc + jnp.dot(cv_ref[...], wc_ref[...], preferred_element_type=jnp.float32)
    out_ref[...] = acc.reshape(_BLK)


def _tc_dense(dec_hidden, context_vector, w_s, w_c):
    return pl.pallas_call(
        _b1_body,
        grid=(B_N // _BLK,),
        in_specs=[
            pl.BlockSpec((_BLK, HID_N), lambda i: (i, 0)),
            pl.BlockSpec((_BLK, HID_N), lambda i: (i, 0)),
            pl.BlockSpec((HID_N, 1), lambda i: (0, 0)),
            pl.BlockSpec((HID_N, 1), lambda i: (0, 0)),
        ],
        out_specs=pl.BlockSpec((_BLK,), lambda i: (i,)),
        out_shape=jax.ShapeDtypeStruct((B_N,), jnp.float32),
    )(dec_hidden, context_vector, w_s, w_c)


def _b2_body(s12_ref, s3_ref, bias_ref, out_ref):
    z = s12_ref[...] + s3_ref[...] + bias_ref[0, 0]
    out_ref[...] = 1.0 / (1.0 + jnp.exp(-z))


def _tc_combine(s12, s3, bias):
    return pl.pallas_call(
        _b2_body,
        grid=(B_N // _BLK,),
        in_specs=[
            pl.BlockSpec((_BLK,), lambda i: (i,)),
            pl.BlockSpec((_BLK,), lambda i: (i,)),
            pl.BlockSpec((1, 1), lambda i: (0, 0), memory_space=pltpu.SMEM),
        ],
        out_specs=pl.BlockSpec((_BLK,), lambda i: (i,)),
        out_shape=jax.ShapeDtypeStruct((B_N,), jnp.float32),
    )(s12, s3, bias)


def kernel(context_vector, dec_hidden, x, embedding_matrix, w_s, b_s, w_c, b_c, w_i, b_i):
    xf = x.reshape(B_N).astype(jnp.int32)
    table_t = embedding_matrix.T
    wbc = jnp.broadcast_to(w_i.reshape(DIM_N, 1), (DIM_N, 16))
    v_sc = _sc_table_dot(table_t, wbc)
    v_tc = _tc_table_dot(table_t, w_i)
    s3 = _sc_gather(v_sc, v_tc, xf)
    s12 = _tc_dense(dec_hidden, context_vector, w_s, w_c)
    bias = (b_s + b_c + b_i).reshape(1, 1)
    return _tc_combine(s12, s3, bias).reshape(B_N, 1)


# speedup vs baseline: 1.0163x; 1.0014x over previous
import jax
import jax.numpy as jnp
from jax import lax
from jax.experimental import pallas as pl
from jax.experimental.pallas import tpu as pltpu
from jax.experimental.pallas import tpu_sc as plsc

VOCAB_N = 1000000
DIM_N = 64
HID_N = 512
B_N = 16384

_NC = 2
_NS = 16
_NW = _NC * _NS
_BPW = B_N // _NW
_NG = _BPW // 16

_SEGS = 8192
_S0 = 3072
_SPT = _S0 // _NW
_CHUNK = 256
_NCHUNK = _SPT * 128 // _CHUNK
_NBUF = 6

_VBLK = 32768
_NVB = (VOCAB_N - _S0 * 128 + _VBLK - 1) // _VBLK
_B0OFF = _S0 * 128 // _VBLK


def _b0_body(tt_ref, wi_ref, out_ref):
    row = jnp.dot(wi_ref[...].T, tt_ref[...], preferred_element_type=jnp.float32)
    out_ref[...] = row.reshape(_VBLK // 128, 128)


def _tc_table_dot(table_t, w_i):
    return pl.pallas_call(
        _b0_body,
        grid=(_NVB,),
        in_specs=[
            pl.BlockSpec((DIM_N, _VBLK), lambda i: (0, i + _B0OFF)),
            pl.BlockSpec((DIM_N, 1), lambda i: (0, 0)),
        ],
        out_specs=pl.BlockSpec((_VBLK // 128, 128), lambda i: (i + _B0OFF, 0)),
        out_shape=jax.ShapeDtypeStruct((_SEGS, 128), jnp.float32),
    )(table_t, w_i)


def _sc1_body(table_t, wbc, vout, bufs_v, wbc_v, stage_v, sems):
    wid = lax.axis_index("s") * _NC + lax.axis_index("c")
    c0 = wid * (_SPT * 128)
    pltpu.sync_copy(wbc, wbc_v)

    for b in range(_NBUF):
        pltpu.async_copy(
            table_t.at[:, pl.ds(c0 + b * _CHUNK, _CHUNK)], bufs_v.at[b], sems.at[b]
        )

    nj = _CHUNK // 16

    def one_chunk(k, b):
        pltpu.make_async_copy(
            table_t.at[:, pl.ds(0, _CHUNK)], bufs_v.at[b], sems.at[b]
        ).wait()

        def cbody(ci, accs):
            c = ci * 2
            wv0 = wbc_v[c, :]
            wv1 = wbc_v[c + 1, :]
            accs = tuple(
                accs[j] + wv0 * bufs_v[b, c, pl.ds(j * 16, 16)] for j in range(nj)
            )
            return tuple(
                accs[j] + wv1 * bufs_v[b, c + 1, pl.ds(j * 16, 16)]
                for j in range(nj)
            )

        accs = lax.fori_loop(
            0, DIM_N // 2, cbody,
            tuple(jnp.zeros((16,), jnp.float32) for _ in range(nj)),
        )
        pltpu.async_copy(
            table_t.at[:, pl.ds(c0 + (k + _NBUF) * _CHUNK, _CHUNK)],
            bufs_v.at[b],
            sems.at[b],
        )
        for j in range(nj):
            row = k * (_CHUNK // 128) + j // 8
            stage_v[row, pl.ds((j % 8) * 16, 16)] = accs[j]

    def chunk_iter(k, carry):
        bdyn = lax.rem(k, _NBUF)
        for b in range(_NBUF):
            @pl.when(bdyn == b)
            def _(b=b):
                one_chunk(k, b)
        return carry

    lax.fori_loop(0, _NCHUNK, chunk_iter, 0)
    for b in range(_NBUF):
        pltpu.make_async_copy(
            table_t.at[:, pl.ds(0, _CHUNK)], bufs_v.at[b], sems.at[b]
        ).wait()
    pltpu.sync_copy(stage_v, vout.at[pl.ds(wid * _SPT, _SPT)])


def _sc_table_dot(table_t, wbc):
    mesh = plsc.VectorSubcoreMesh(core_axis_name="c", subcore_axis_name="s")
    return pl.kernel(
        _sc1_body,
        mesh=mesh,
        out_type=jax.ShapeDtypeStruct((_SEGS, 128), jnp.float32),
        scratch_types=[
            pltpu.VMEM((_NBUF, DIM_N, _CHUNK), jnp.float32),
            pltpu.VMEM((DIM_N, 16), jnp.float32),
            pltpu.VMEM((_SPT, 128), jnp.float32),
            pltpu.SemaphoreType.DMA((_NBUF,)),
        ],
        compiler_params=pltpu.CompilerParams(needs_layout_passes=False),
    )(table_t, wbc)


def _sc2_body(va, vb, xf, out, idx_v, seg_v, lane_v, da_v, db_v, s3_v, sem):
    wid = lax.axis_index("s") * _NC + lax.axis_index("c")
    rbase = wid * _BPW
    pltpu.sync_copy(xf.at[pl.ds(rbase, _BPW)], idx_v)

    def split(g, carry):
        r = idx_v[pl.ds(g * 16, 16)]
        seg_v[pl.ds(g * 16, 16)] = r >> 7
        lane_v[pl.ds(g * 16, 16)] = r & 127
        return carry

    lax.fori_loop(0, _NG, split, 0)

    lanes = lax.iota(jnp.int32, 16)
    for h in range(2):
        half = h * (_BPW // 2)
        cpa = pltpu.async_copy(va.at[seg_v.at[pl.ds(half, _BPW // 2)]], da_v, sem)
        cpb = pltpu.async_copy(vb.at[seg_v.at[pl.ds(half, _BPW // 2)]], db_v, sem)
        cpa.wait()
        cpb.wait()

        def extract(g, carry):
            rows = g * 16 + lanes
            cols = lane_v[pl.ds(half + g * 16, 16)]
            a = plsc.load_gather(da_v, [rows, cols])
            b = plsc.load_gather(db_v, [rows, cols])
            seg = seg_v[pl.ds(half + g * 16, 16)]
            s3_v[pl.ds(half + g * 16, 16)] = jnp.where(seg < _S0, a, b)
            return carry

        lax.fori_loop(0, _NG // 2, extract, 0)

    pltpu.sync_copy(s3_v, out.at[pl.ds(rbase, _BPW)])


def _sc_gather(va, vb, xf):
    mesh = plsc.VectorSubcoreMesh(core_axis_name="c", subcore_axis_name="s")
    return pl.kernel(
        _sc2_body,
        mesh=mesh,
        out_type=jax.ShapeDtypeStruct((B_N,), jnp.float32),
        scratch_types=[
            pltpu.VMEM((_BPW,), jnp.int32),
            pltpu.VMEM((_BPW,), jnp.int32),
            pltpu.VMEM((_BPW,), jnp.int32),
            pltpu.VMEM((_BPW // 2, 128), jnp.float32),
            pltpu.VMEM((_BPW // 2, 128), jnp.float32),
            pltpu.VMEM((_BPW,), jnp.float32),
            pltpu.SemaphoreType.DMA,
        ],
        compiler_params=pltpu.CompilerParams(needs_layout_passes=False),
    )(va, vb, xf)


_BLK = 2048


def _b1_body(dh_ref, cv_ref, ws_ref, wc_ref, out_ref):
    acc = jnp.dot(dh_ref[...], ws_ref[...], preferred_element_type=jnp.float32)
    acc = acc + jnp.dot(cv_ref[...], wc_ref[...], preferred_element_type=jnp.float32)
    out_ref[...] = acc.reshape(_BLK)


def _tc_dense(dec_hidden, context_vector, w_s, w_c):
    return pl.pallas_call(
        _b1_body,
        grid=(B_N // _BLK,),
        in_specs=[
            pl.BlockSpec((_BLK, HID_N), lambda i: (i, 0)),
            pl.BlockSpec((_BLK, HID_N), lambda i: (i, 0)),
            pl.BlockSpec((HID_N, 1), lambda i: (0, 0)),
            pl.BlockSpec((HID_N, 1), lambda i: (0, 0)),
        ],
        out_specs=pl.BlockSpec((_BLK,), lambda i: (i,)),
        out_shape=jax.ShapeDtypeStruct((B_N,), jnp.float32),
    )(dec_hidden, context_vector, w_s, w_c)


def _b2_body(s12_ref, s3_ref, bias_ref, out_ref):
    z = s12_ref[...] + s3_ref[...] + bias_ref[0, 0]
    out_ref[...] = 1.0 / (1.0 + jnp.exp(-z))


def _tc_combine(s12, s3, bias):
    return pl.pallas_call(
        _b2_body,
        grid=(B_N // _BLK,),
        in_specs=[
            pl.BlockSpec((_BLK,), lambda i: (i,)),
            pl.BlockSpec((_BLK,), lambda i: (i,)),
            pl.BlockSpec((1, 1), lambda i: (0, 0), memory_space=pltpu.SMEM),
        ],
        out_specs=pl.BlockSpec((_BLK,), lambda i: (i,)),
        out_shape=jax.ShapeDtypeStruct((B_N,), jnp.float32),
    )(s12, s3, bias)


def kernel(context_vector, dec_hidden, x, embedding_matrix, w_s, b_s, w_c, b_c, w_i, b_i):
    xf = x.reshape(B_N).astype(jnp.int32)
    table_t = embedding_matrix.T
    wbc = jnp.broadcast_to(w_i.reshape(DIM_N, 1), (DIM_N, 16))
    v_sc = _sc_table_dot(table_t, wbc)
    v_tc = _tc_table_dot(table_t, w_i)
    s3 = _sc_gather(v_sc, v_tc, xf)
    s12 = _tc_dense(dec_hidden, context_vector, w_s, w_c)
    bias = (b_s + b_c + b_i).reshape(1, 1)
    return _tc_combine(s12, s3, bias).reshape(B_N, 1)


# speedup vs baseline: 1.1272x; 1.1091x over previous
import jax
import jax.numpy as jnp
from jax import lax
from jax.experimental import pallas as pl
from jax.experimental.pallas import tpu as pltpu
from jax.experimental.pallas import tpu_sc as plsc

VOCAB_N = 1000000
DIM_N = 64
HID_N = 512
B_N = 16384

_NC = 2
_NS = 16
_NW = _NC * _NS
_BPW = B_N // _NW
_NG = _BPW // 16

_SEGS = 8192
_VBLK = 32768
_NVB = (VOCAB_N + _VBLK - 1) // _VBLK


def _b0_body(tt_ref, wi_ref, out_ref):
    row = jnp.dot(wi_ref[...].T, tt_ref[...], preferred_element_type=jnp.float32)
    out_ref[...] = row.reshape(_VBLK // 128, 128)


def _tc_table_dot(table_t, w_i):
    return pl.pallas_call(
        _b0_body,
        grid=(_NVB,),
        in_specs=[
            pl.BlockSpec((DIM_N, _VBLK), lambda i: (0, i)),
            pl.BlockSpec((DIM_N, 1), lambda i: (0, 0)),
        ],
        out_specs=pl.BlockSpec((_VBLK // 128, 128), lambda i: (i, 0)),
        out_shape=jax.ShapeDtypeStruct((_SEGS, 128), jnp.float32),
    )(table_t, w_i)


def _sc_body(v, xf, out, idx_v, seg_v, lane_v, data_v, s3_v, sem):
    wid = lax.axis_index("s") * _NC + lax.axis_index("c")
    rbase = wid * _BPW
    pltpu.sync_copy(xf.at[pl.ds(rbase, _BPW)], idx_v)

    def split(g, carry):
        r = idx_v[pl.ds(g * 16, 16)]
        seg_v[pl.ds(g * 16, 16)] = r >> 7
        lane_v[pl.ds(g * 16, 16)] = r & 127
        return carry

    lax.fori_loop(0, _NG, split, 0)

    pltpu.async_copy(v.at[seg_v], data_v, sem).wait()

    lanes = lax.iota(jnp.int32, 16)

    def extract(g, carry):
        rows = g * 16 + lanes
        cols = lane_v[pl.ds(g * 16, 16)]
        s3_v[pl.ds(g * 16, 16)] = plsc.load_gather(data_v, [rows, cols])
        return carry

    lax.fori_loop(0, _NG, extract, 0)
    pltpu.sync_copy(s3_v, out.at[pl.ds(rbase, _BPW)])


def _sc_gather(v, xf):
    mesh = plsc.VectorSubcoreMesh(core_axis_name="c", subcore_axis_name="s")
    return pl.kernel(
        _sc_body,
        mesh=mesh,
        out_type=jax.ShapeDtypeStruct((B_N,), jnp.float32),
        scratch_types=[
            pltpu.VMEM((_BPW,), jnp.int32),
            pltpu.VMEM((_BPW,), jnp.int32),
            pltpu.VMEM((_BPW,), jnp.int32),
            pltpu.VMEM((_BPW, 128), jnp.float32),
            pltpu.VMEM((_BPW,), jnp.float32),
            pltpu.SemaphoreType.DMA,
        ],
        compiler_params=pltpu.CompilerParams(needs_layout_passes=False),
    )(v, xf)


_BLK = 4096


def _b1_body(dh_ref, cv_ref, ws_ref, wc_ref, out_ref):
    acc = jnp.dot(dh_ref[...], ws_ref[...], preferred_element_type=jnp.float32)
    acc = acc + jnp.dot(cv_ref[...], wc_ref[...], preferred_element_type=jnp.float32)
    out_ref[...] = acc.reshape(_BLK)


def _tc_dense(dec_hidden, context_vector, w_s, w_c):
    return pl.pallas_call(
        _b1_body,
        grid=(B_N // _BLK,),
        in_specs=[
            pl.BlockSpec((_BLK, HID_N), lambda i: (i, 0)),
            pl.BlockSpec((_BLK, HID_N), lambda i: (i, 0)),
            pl.BlockSpec((HID_N, 1), lambda i: (0, 0)),
            pl.BlockSpec((HID_N, 1), lambda i: (0, 0)),
        ],
        out_specs=pl.BlockSpec((_BLK,), lambda i: (i,)),
        out_shape=jax.ShapeDtypeStruct((B_N,), jnp.float32),
    )(dec_hidden, context_vector, w_s, w_c)


def _b2_body(s12_ref, s3_ref, bias_ref, out_ref):
    z = s12_ref[...] + s3_ref[...] + bias_ref[0, 0]
    out_ref[...] = 1.0 / (1.0 + jnp.exp(-z))


def _tc_combine(s12, s3, bias):
    return pl.pallas_call(
        _b2_body,
        grid=(B_N // _BLK,),
        in_specs=[
            pl.BlockSpec((_BLK,), lambda i: (i,)),
            pl.BlockSpec((_BLK,), lambda i: (i,)),
            pl.BlockSpec((1, 1), lambda i: (0, 0), memory_space=pltpu.SMEM),
        ],
        out_specs=pl.BlockSpec((_BLK,), lambda i: (i,)),
        out_shape=jax.ShapeDtypeStruct((B_N,), jnp.float32),
    )(s12, s3, bias)


def kernel(context_vector, dec_hidden, x, embedding_matrix, w_s, b_s, w_c, b_c, w_i, b_i):
    xf = x.reshape(B_N).astype(jnp.int32)
    table_t = embedding_matrix.T
    v = _tc_table_dot(table_t, w_i)
    s3 = _sc_gather(v, xf)
    s12 = _tc_dense(dec_hidden, context_vector, w_s, w_c)
    bias = (b_s + b_c + b_i).reshape(1, 1)
    return _tc_combine(s12, s3, bias).reshape(B_N, 1)
